# Initial kernel scaffold; baseline (speedup 1.0000x reference)
import functools

import jax
import jax.numpy as jnp
from jax import lax
from jax.experimental import pallas as pl
from jax.experimental.pallas import tpu as pltpu
from jax.experimental.pallas import tpu_sc as plsc

EMBED_DIM = 64
BATCH = 4096
SEQ_LEN = 200

_info = plsc.get_sparse_core_info()
NC, NS, L = _info.num_cores, _info.num_subcores, _info.num_lanes
NW = NC * NS

ROWS_PER_W = BATCH // NW
GROUP_ROWS = 16
NGROUPS = ROWS_PER_W // GROUP_ROWS
GROUP_TOK = GROUP_ROWS * SEQ_LEN
GCHUNK = 128
NCHUNKS = GROUP_TOK // GCHUNK
NFULL = SEQ_LEN // L
TAIL_OFF = SEQ_LEN - L


def _sc_kernel(inp_hbm, w_hbm, out_hbm, inp_v, idx_v, rows_v, sem):
    wid = lax.axis_index("s") * NC + lax.axis_index("c")
    row0 = wid * ROWS_PER_W

    def group_body(g, _):
        rbase = row0 + g * GROUP_ROWS
        pltpu.sync_copy(inp_hbm.at[pl.ds(rbase, GROUP_ROWS), :], inp_v)

        def row_body(r, _):
            base = pl.multiple_of(r * SEQ_LEN, 8)
            carry = jnp.int32(0)
            for k in range(NFULL):
                x = inp_v[r, pl.ds(k * L, L)]
                m = (x != 0).astype(jnp.int32)
                c = plsc.cumsum(m)
                idx_v[pl.ds(base + k * L, L)] = (c + carry) * m
                carry = carry + jnp.max(c)
            x = inp_v[r, pl.ds(TAIL_OFF, L)]
            m = (x != 0).astype(jnp.int32)
            c = plsc.cumsum(m)
            lane = lax.iota(jnp.int32, L)
            c7 = jnp.max(jnp.where(lane < (L - (SEQ_LEN - NFULL * L)), c, 0))
            idx_v[pl.ds(base + TAIL_OFF, L)] = (carry - c7 + c) * m
            return 0

        lax.fori_loop(0, GROUP_ROWS, row_body, 0)

        tok0 = pl.multiple_of(rbase * SEQ_LEN, 8)

        def chunk_body(ci, _):
            off = pl.multiple_of(ci * GCHUNK, 8)
            pltpu.async_copy(
                w_hbm.at[idx_v.at[pl.ds(off, GCHUNK)]], rows_v, sem
            ).wait()
            pltpu.sync_copy(rows_v, out_hbm.at[pl.ds(tok0 + off, GCHUNK), :])
            return 0

        lax.fori_loop(0, NCHUNKS, chunk_body, 0)
        return 0

    lax.fori_loop(0, NGROUPS, group_body, 0)


@functools.partial(
    pl.kernel,
    out_type=jax.ShapeDtypeStruct((BATCH * SEQ_LEN, EMBED_DIM), jnp.float32),
    mesh=plsc.VectorSubcoreMesh(core_axis_name="c", subcore_axis_name="s"),
    scratch_types=[
        pltpu.VMEM((GROUP_ROWS, SEQ_LEN), jnp.int32),
        pltpu.VMEM((GROUP_TOK,), jnp.int32),
        pltpu.VMEM((GCHUNK, EMBED_DIM), jnp.float32),
        pltpu.SemaphoreType.DMA,
    ],
)
def _embed_lookup(inp_hbm, w_hbm, out_hbm, inp_v, idx_v, rows_v, sem):
    _sc_kernel(inp_hbm, w_hbm, out_hbm, inp_v, idx_v, rows_v, sem)


def kernel(input, weights):
    inp = input.astype(jnp.int32)
    out = _embed_lookup(inp, weights.astype(jnp.float32))
    return out.reshape(BATCH, SEQ_LEN, EMBED_DIM)


# speedup vs baseline: 2.7765x; 2.7765x over previous
import functools

import jax
import jax.numpy as jnp
from jax import lax
from jax.experimental import pallas as pl
from jax.experimental.pallas import tpu as pltpu
from jax.experimental.pallas import tpu_sc as plsc

EMBED_DIM = 64
BATCH = 4096
SEQ_LEN = 200

_info = plsc.get_sparse_core_info()
NC, NS, L = _info.num_cores, _info.num_subcores, _info.num_lanes
NW = NC * NS

ROWS_PER_W = BATCH // NW
GROUP_ROWS = 16
NGROUPS = ROWS_PER_W // GROUP_ROWS
GROUP_TOK = GROUP_ROWS * SEQ_LEN
GCHUNK = 128
NCHUNKS = GROUP_TOK // GCHUNK
NFULL = SEQ_LEN // L
TAIL_OFF = SEQ_LEN - L


def _sc_kernel(inp_hbm, w_hbm, out_hbm, inp_v, idx_v, rows_v, sem):
    wid = lax.axis_index("s") * NC + lax.axis_index("c")
    row0 = wid * ROWS_PER_W

    def group_body(g, _):
        rbase = row0 + g * GROUP_ROWS
        pltpu.sync_copy(inp_hbm.at[pl.ds(rbase, GROUP_ROWS), :], inp_v)

        def row_body(r, _):
            base = pl.multiple_of(r * SEQ_LEN, 8)
            carry = jnp.int32(0)
            for k in range(NFULL):
                x = inp_v[r, pl.ds(k * L, L)]
                m = jnp.minimum(jnp.abs(x), 1)
                c = plsc.cumsum(m)
                idx_v[pl.ds(base + k * L, L)] = (c + carry) * m
                carry = carry + jnp.max(c)
            x = inp_v[r, pl.ds(TAIL_OFF, L)]
            m = jnp.minimum(jnp.abs(x), 1)
            c = plsc.cumsum(m)
            lane = lax.iota(jnp.int32, L)
            nlap = L - (SEQ_LEN - NFULL * L)
            c7 = jnp.max(jnp.where(lane < nlap, c, 0))
            idx_v[pl.ds(base + TAIL_OFF, L)] = (carry - c7 + c) * m
            return 0

        lax.fori_loop(0, GROUP_ROWS, row_body, 0)

        tok0 = pl.multiple_of(rbase * SEQ_LEN, 8)

        def chunk_body(ci, _):
            off = pl.multiple_of(ci * GCHUNK, 8)
            pltpu.async_copy(
                w_hbm.at[idx_v.at[pl.ds(off, GCHUNK)]], rows_v, sem
            ).wait()
            pltpu.sync_copy(rows_v, out_hbm.at[pl.ds(tok0 + off, GCHUNK), :])
            return 0

        lax.fori_loop(0, NCHUNKS, chunk_body, 0)
        return 0

    lax.fori_loop(0, NGROUPS, group_body, 0)


@functools.partial(
    pl.kernel,
    out_type=jax.ShapeDtypeStruct((BATCH * SEQ_LEN, EMBED_DIM), jnp.float32),
    mesh=plsc.VectorSubcoreMesh(core_axis_name="c", subcore_axis_name="s"),
    scratch_types=[
        pltpu.VMEM((GROUP_ROWS, SEQ_LEN), jnp.int32),
        pltpu.VMEM((GROUP_TOK,), jnp.int32),
        pltpu.VMEM((GCHUNK, EMBED_DIM), jnp.float32),
        pltpu.SemaphoreType.DMA,
    ],
    compiler_params=pltpu.CompilerParams(
        use_tc_tiling_on_sc=False, needs_layout_passes=False
    ),
)
def _embed_lookup(inp_hbm, w_hbm, out_hbm, inp_v, idx_v, rows_v, sem):
    _sc_kernel(inp_hbm, w_hbm, out_hbm, inp_v, idx_v, rows_v, sem)


def kernel(input, weights):
    inp = input.astype(jnp.int32)
    out = _embed_lookup(inp, weights.astype(jnp.float32))
    return out.reshape(BATCH, SEQ_LEN, EMBED_DIM)


# speedup vs baseline: 2.8121x; 1.0128x over previous
import functools

import jax
import jax.numpy as jnp
from jax import lax
from jax.experimental import pallas as pl
from jax.experimental.pallas import tpu as pltpu
from jax.experimental.pallas import tpu_sc as plsc

EMBED_DIM = 64
BATCH = 4096
SEQ_LEN = 200

_info = plsc.get_sparse_core_info()
NC, NS, L = _info.num_cores, _info.num_subcores, _info.num_lanes
NW = NC * NS

ROWS_PER_W = BATCH // NW
TOK_PER_W = ROWS_PER_W * SEQ_LEN
GCHUNK = 128
NCHUNKS = TOK_PER_W // GCHUNK
NBUF = 4
NITER = NCHUNKS // NBUF
NFULL = SEQ_LEN // L
TAIL_OFF = SEQ_LEN - L


def _sc_body(inp_hbm, w_hbm, out_hbm, inp_v, idx_v, rows_v, gsem, ssem):
    wid = lax.axis_index("s") * NC + lax.axis_index("c")
    row0 = wid * ROWS_PER_W
    tok0 = pl.multiple_of(row0 * SEQ_LEN, 8)

    pltpu.sync_copy(inp_hbm.at[pl.ds(row0, ROWS_PER_W), :], inp_v)

    def row_body(r, _):
        base = pl.multiple_of(r * SEQ_LEN, 8)
        carry = jnp.int32(0)
        for k in range(NFULL):
            x = inp_v[r, pl.ds(k * L, L)]
            m = jnp.minimum(jnp.abs(x), 1)
            c = plsc.cumsum(m)
            idx_v[pl.ds(base + k * L, L)] = (c + carry) * m
            carry = carry + jnp.max(c)
        x = inp_v[r, pl.ds(TAIL_OFF, L)]
        m = jnp.minimum(jnp.abs(x), 1)
        c = plsc.cumsum(m)
        lane = lax.iota(jnp.int32, L)
        nlap = L - (SEQ_LEN - NFULL * L)
        c7 = jnp.max(jnp.where(lane < nlap, c, 0))
        idx_v[pl.ds(base + TAIL_OFF, L)] = (carry - c7 + c) * m
        return 0

    lax.fori_loop(0, ROWS_PER_W, row_body, 0)

    def gather_desc(b, off):
        return pltpu.make_async_copy(
            w_hbm.at[idx_v.at[pl.ds(off, GCHUNK)]], rows_v.at[b], gsem.at[b]
        )

    def scatter_desc(b, off):
        return pltpu.make_async_copy(
            rows_v.at[b], out_hbm.at[pl.ds(tok0 + off, GCHUNK), :], ssem.at[b]
        )

    for b in range(NBUF):
        gather_desc(b, b * GCHUNK).start()

    @pl.loop(0, NITER)
    def ring(i):
        c0 = i * NBUF
        for b in range(NBUF):
            off = pl.multiple_of((c0 + b) * GCHUNK, 8)
            gather_desc(b, off).wait()
            scatter_desc(b, off).start()
        for b in range(NBUF):
            off = pl.multiple_of((c0 + b) * GCHUNK, 8)
            scatter_desc(b, off).wait()

            @pl.when(i < NITER - 1)
            def _():
                noff = pl.multiple_of((c0 + NBUF + b) * GCHUNK, 8)
                gather_desc(b, noff).start()


@functools.partial(
    pl.kernel,
    out_type=jax.ShapeDtypeStruct((BATCH * SEQ_LEN, EMBED_DIM), jnp.float32),
    mesh=plsc.VectorSubcoreMesh(core_axis_name="c", subcore_axis_name="s"),
    scratch_types=[
        pltpu.VMEM((ROWS_PER_W, SEQ_LEN), jnp.int32),
        pltpu.VMEM((TOK_PER_W,), jnp.int32),
        pltpu.VMEM((NBUF, GCHUNK, EMBED_DIM), jnp.float32),
        pltpu.SemaphoreType.DMA((NBUF,)),
        pltpu.SemaphoreType.DMA((NBUF,)),
    ],
    compiler_params=pltpu.CompilerParams(
        use_tc_tiling_on_sc=False, needs_layout_passes=False
    ),
)
def _embed_lookup(inp_hbm, w_hbm, out_hbm, inp_v, idx_v, rows_v, gsem, ssem):
    _sc_body(inp_hbm, w_hbm, out_hbm, inp_v, idx_v, rows_v, gsem, ssem)


def kernel(input, weights):
    inp = input.astype(jnp.int32)
    out = _embed_lookup(inp, weights.astype(jnp.float32))
    return out.reshape(BATCH, SEQ_LEN, EMBED_DIM)
